# baseline (speedup 1.0000x reference)
import functools

import jax
import jax.numpy as jnp
from jax import lax
from jax.experimental import pallas as pl
from jax.experimental.pallas import tpu as pltpu
from jax.experimental.pallas import tpu_sc as plsc

N_NODES = 50000
N_EDGES = 800000
HID = 64
NP = 50048
NC, NS = 2, 16
NW = NC * NS
CH = 128
EPT = 25600
R = 400
NCHK = EPT // R
EPAD = EPT * NW
BR = 2176
GR = NP // BR


@functools.lru_cache(None)
def _make_apply(C):
    mesh = plsc.VectorSubcoreMesh(core_axis_name="c", subcore_axis_name="s")

    nzt = NP // CH

    @functools.partial(
        pl.kernel,
        out_type=jax.ShapeDtypeStruct((NC, NP, C), jnp.float32),
        mesh=mesh,
        scratch_types=[
            pltpu.VMEM((R,), jnp.int32),
            pltpu.VMEM((R,), jnp.int32),
            pltpu.VMEM((R,), jnp.int32),
            pltpu.VMEM((R,), jnp.int32),
            pltpu.VMEM((R, C), jnp.float32),
            pltpu.VMEM((R, C), jnp.float32),
            pltpu.VMEM_SHARED((NP, C), jnp.float32),
        ] + [pltpu.SemaphoreType.DMA] * 4,
        compiler_params=pltpu.CompilerParams(use_tc_tiling_on_sc=False),
    )
    def apply_k(tab, gidx, sidx, ztile, out, gv0, gv1, sv0, sv1, rb0, rb1,
                acc, g0, g1, t0, t1):
        gvs = (gv0, gv1)
        svs = (sv0, sv1)
        rbs = (rb0, rb1)
        gsem = (g0, g1)
        ssem = (t0, t1)
        cid = lax.axis_index("c")
        sid = lax.axis_index("s")
        wid = sid * NC + cid

        def zero_body(i, carry):
            t = i * NS + sid

            @pl.when(t < nzt)
            def _():
                pltpu.sync_copy(ztile, acc.at[pl.ds(t * CH, CH)])

            return carry

        lax.fori_loop(0, (nzt + NS - 1) // NS, zero_body, 0, unroll=False)
        plsc.subcore_barrier()

        pltpu.sync_copy(gidx.at[wid].at[pl.ds(0, R)], gvs[0])
        pltpu.sync_copy(sidx.at[wid].at[pl.ds(0, R)], svs[0])
        pltpu.async_copy(tab.at[gvs[0]], rbs[0], gsem[0])

        def body(i, c2):
            for p in range(2):
                c = i * 2 + p
                cn = c + 1
                nxt = 1 - p

                @pl.when((cn < NCHK) & (c >= 1))
                def _drain(nxt=nxt):
                    pltpu.make_async_copy(
                        rbs[nxt], acc.at[svs[nxt]], ssem[nxt]).wait()

                @pl.when(cn < NCHK)
                def _pref(nxt=nxt, cn=cn):
                    pltpu.sync_copy(gidx.at[wid].at[pl.ds(cn * R, R)],
                                    gvs[nxt])
                    pltpu.sync_copy(sidx.at[wid].at[pl.ds(cn * R, R)],
                                    svs[nxt])
                    pltpu.async_copy(tab.at[gvs[nxt]], rbs[nxt], gsem[nxt])

                pltpu.make_async_copy(tab.at[gvs[p]], rbs[p], gsem[p]).wait()
                pltpu.async_copy(rbs[p], acc.at[svs[p]], ssem[p], add=True)

            return c2

        lax.fori_loop(0, NCHK // 2, body, 0, unroll=False)
        for p in range(2):
            pltpu.make_async_copy(rbs[p], acc.at[svs[p]], ssem[p]).wait()
        plsc.subcore_barrier()

        @pl.when(sid == 0)
        def _dump():
            pltpu.sync_copy(acc, out.at[cid])

    return apply_k


def _prep(degp, x0p):
    def body(deg_ref, x0_ref, dinv_ref, yt_ref):
        deg = deg_ref[0, :, 0:1] + deg_ref[1, :, 0:1]
        dinv = jnp.where(deg > 0, lax.rsqrt(deg), 0.0)
        dinv_ref[...] = dinv
        yt_ref[...] = x0_ref[...] * dinv

    return pl.pallas_call(
        body,
        grid=(GR,),
        in_specs=[
            pl.BlockSpec((NC, BR, 32), lambda i: (0, i, 0)),
            pl.BlockSpec((BR, 32), lambda i: (i, 0)),
        ],
        out_specs=[
            pl.BlockSpec((BR, 1), lambda i: (i, 0)),
            pl.BlockSpec((BR, 32), lambda i: (i, 0)),
        ],
        out_shape=[
            jax.ShapeDtypeStruct((NP, 1), jnp.float32),
            jax.ShapeDtypeStruct((NP, 32), jnp.float32),
        ],
    )(degp, x0p)


def _cheb_step(partials, dinv, prev2):
    nch = len(partials)
    first = prev2 is None
    Cs = [p.shape[-1] for p in partials]

    def body(*refs):
        prefs = refs[:nch]
        dref = refs[nch]
        p2refs = () if first else refs[nch + 1: nch + 1 + nch]
        orefs = refs[nch + 1 + (0 if first else nch):]
        txrefs = orefs[:nch]
        ytrefs = orefs[nch:]
        dv = dref[...]
        for q in range(nch):
            ltx = -(prefs[q][0] + prefs[q][1]) * dv
            tx = ltx if first else 2.0 * ltx - p2refs[q][...]
            txrefs[q][...] = tx
            ytrefs[q][...] = tx * dv

    in_specs = [pl.BlockSpec((NC, BR, C), lambda i: (0, i, 0)) for C in Cs]
    in_specs.append(pl.BlockSpec((BR, 1), lambda i: (i, 0)))
    args = list(partials) + [dinv]
    if not first:
        in_specs += [pl.BlockSpec((BR, C), lambda i: (i, 0)) for C in Cs]
        args += list(prev2)
    out_specs = [pl.BlockSpec((BR, C), lambda i: (i, 0)) for C in Cs] * 2
    out_shape = [jax.ShapeDtypeStruct((NP, C), jnp.float32) for C in Cs] * 2

    outs = pl.pallas_call(
        body, grid=(GR,), in_specs=in_specs, out_specs=out_specs,
        out_shape=out_shape,
    )(*args)
    return list(outs[:nch]), list(outs[nch:])


def _layer_mm(tx_list, W, dinv):
    K = W.shape[0]
    nch = len(tx_list[0])
    Cs = [t.shape[-1] for t in tx_list[0]]
    F = W.shape[1]

    def body(*refs):
        nin = K * nch
        xrefs = refs[:nin]
        wref = refs[nin]
        dref = refs[nin + 1]
        oA, oB, yA, yB = refs[nin + 2: nin + 6]
        acc = jnp.zeros((BR, HID), jnp.float32)
        for k in range(K):
            off = 0
            for q in range(nch):
                xb = xrefs[k * nch + q][...]
                acc = acc + jnp.dot(
                    xb, wref[k, off:off + Cs[q], :],
                    preferred_element_type=jnp.float32)
                off += Cs[q]
        o = jnp.where(acc > 0, acc, 0.5 * acc)
        dv = dref[...]
        oA[...] = o[:, :32]
        oB[...] = o[:, 32:]
        yA[...] = o[:, :32] * dv
        yB[...] = o[:, 32:] * dv

    in_specs = [pl.BlockSpec((BR, C), lambda i: (i, 0))
                for _k in range(K) for C in Cs]
    in_specs.append(pl.BlockSpec((K, F, HID), lambda i: (0, 0, 0)))
    in_specs.append(pl.BlockSpec((BR, 1), lambda i: (i, 0)))
    out_specs = [pl.BlockSpec((BR, 32), lambda i: (i, 0))] * 4
    out_shape = [jax.ShapeDtypeStruct((NP, 32), jnp.float32)] * 4
    args = [t for txs in tx_list for t in txs] + [W, dinv]
    o = pl.pallas_call(
        body, grid=(GR,), in_specs=in_specs, out_specs=out_specs,
        out_shape=out_shape,
    )(*args)
    return [o[0], o[1]], [o[2], o[3]]


def _final(outs, x0p):
    def body(*refs):
        (a1, b1, a2, b2, a3, b3, a4, b4, a5, b5, a6, b6, xref, gref) = refs
        i = pl.program_id(0)

        @pl.when(i == 0)
        def _init():
            gref[...] = jnp.zeros_like(gref)

        xb = xref[...]
        pairs = [(a1, a4), (b1, b4), (a2, a5), (b2, b5), (a3, a6), (b3, b6)]
        accs = []
        for (a, b) in pairs:
            h = a[...] + b[...]
            accs.append(lax.dot_general(
                h, xb, (((0,), (0,)), ((), ())),
                preferred_element_type=jnp.float32))
        g = gref[...] + jnp.concatenate(accs, axis=0)

        @pl.when(i < GR - 1)
        def _store():
            gref[...] = g

        @pl.when(i == GR - 1)
        def _done():
            gref[...] = g * lax.rsqrt(jnp.sum(g * g, axis=1, keepdims=True))

    in_specs = [pl.BlockSpec((BR, 32), lambda i: (i, 0))] * 12
    in_specs.append(pl.BlockSpec((BR, 32), lambda i: (i, 0)))
    return pl.pallas_call(
        body, grid=(GR,), in_specs=in_specs,
        out_specs=pl.BlockSpec((192, 32), lambda i: (0, 0)),
        out_shape=jax.ShapeDtypeStruct((192, 32), jnp.float32),
    )(*outs, x0p)


def _cheb_layer(x_chunks, yt_chunks, W, dinv, gidx, sidx, ztile):
    K = W.shape[0]
    Txs = [list(x_chunks)]
    yt = list(yt_chunks)
    for k in range(1, K):
        partials = [_make_apply(t.shape[-1])(t, gidx, sidx, ztile)
                    for t in yt]
        prev2 = None if k == 1 else Txs[k - 2]
        tx, yt = _cheb_step(partials, dinv, prev2)
        Txs.append(tx)
    return _layer_mm(Txs, W, dinv)


def kernel(pos, x, batch, edge_index, W1, W2, W3, W4, W5, W6):
    src = edge_index[0].astype(jnp.int32)
    dst = edge_index[1].astype(jnp.int32)

    srcg = jnp.where(src == dst, N_NODES, src)
    padg = jnp.full((EPAD - N_EDGES,), N_NODES, jnp.int32)
    pads = jnp.zeros((EPAD - N_EDGES,), jnp.int32)
    gidx = jnp.concatenate([srcg, padg]).reshape(NW, EPT)
    sidx_dst = jnp.concatenate([dst, pads]).reshape(NW, EPT)
    sidx_src = jnp.concatenate([src, pads]).reshape(NW, EPT)

    x0 = jnp.concatenate([pos, x], axis=1).astype(jnp.float32)
    x0p = jnp.pad(x0, ((0, NP - N_NODES), (0, 32 - 9)))
    e0 = jnp.pad(jnp.ones((N_NODES, 1), jnp.float32),
                 ((0, NP - N_NODES), (0, 31)))
    ztile = jnp.zeros((CH, 32), jnp.float32)
    W1p = jnp.pad(W1, ((0, 0), (0, 23), (0, 0)))
    W4p = jnp.pad(W4, ((0, 0), (0, 23), (0, 0)))

    degp = _make_apply(32)(e0, gidx, sidx_src, ztile)
    dinv, ytab0 = _prep(degp, x0p)

    run = functools.partial(_cheb_layer, dinv=dinv, gidx=gidx,
                            sidx=sidx_dst, ztile=ztile)
    out1, yt1 = run([x0p], [ytab0], W1p)
    out2, yt2 = run(out1, yt1, W2)
    out3, _ = run(out2, yt2, W3)
    out4, yt4 = run([x0p], [ytab0], W4p)
    out5, yt5 = run(out4, yt4, W5)
    out6, _ = run(out5, yt5, W6)

    G = _final(out1 + out2 + out3 + out4 + out5 + out6, x0p)
    return G[:, :9].reshape(1, 192, 9)


# speedup vs baseline: 1.7873x; 1.7873x over previous
import functools

import jax
import jax.numpy as jnp
from jax import lax
from jax.experimental import pallas as pl
from jax.experimental.pallas import tpu as pltpu
from jax.experimental.pallas import tpu_sc as plsc

N_NODES = 50000
N_EDGES = 800000
HID = 64
NP = 50048
NC, NS = 2, 16
NW = NC * NS
CH = 128
EPT = 25600
R = 400
NCHK = EPT // R
EPAD = EPT * NW
BR = 2176
GR = NP // BR


@functools.lru_cache(None)
def _make_apply(C):
    mesh = plsc.VectorSubcoreMesh(core_axis_name="c", subcore_axis_name="s")

    nzt = NP // CH

    @functools.partial(
        pl.kernel,
        out_type=jax.ShapeDtypeStruct((NC, NP, C), jnp.float32),
        mesh=mesh,
        scratch_types=[
            pltpu.VMEM((R,), jnp.int32),
            pltpu.VMEM((R,), jnp.int32),
            pltpu.VMEM((R,), jnp.int32),
            pltpu.VMEM((R,), jnp.int32),
            pltpu.VMEM((R, C), jnp.float32),
            pltpu.VMEM((R, C), jnp.float32),
            pltpu.VMEM_SHARED((NP, C), jnp.float32),
        ] + [pltpu.SemaphoreType.DMA] * 4,
        compiler_params=pltpu.CompilerParams(use_tc_tiling_on_sc=False),
    )
    def apply_k(tab, gidx, sidx, ztile, out, gv0, gv1, sv0, sv1, rb0, rb1,
                acc, g0, g1, t0, t1):
        gvs = (gv0, gv1)
        svs = (sv0, sv1)
        rbs = (rb0, rb1)
        gsem = (g0, g1)
        ssem = (t0, t1)
        cid = lax.axis_index("c")
        sid = lax.axis_index("s")
        wid = sid * NC + cid

        def zero_body(i, carry):
            t = i * NS + sid

            @pl.when(t < nzt)
            def _():
                pltpu.sync_copy(ztile, acc.at[pl.ds(t * CH, CH)])

            return carry

        lax.fori_loop(0, (nzt + NS - 1) // NS, zero_body, 0, unroll=False)
        plsc.subcore_barrier()

        pltpu.sync_copy(gidx.at[wid].at[pl.ds(0, R)], gvs[0])
        pltpu.sync_copy(sidx.at[wid].at[pl.ds(0, R)], svs[0])
        pltpu.async_copy(tab.at[gvs[0]], rbs[0], gsem[0])

        def body(i, c2):
            for p in range(2):
                c = i * 2 + p
                cn = c + 1
                nxt = 1 - p

                @pl.when((cn < NCHK) & (c >= 1))
                def _drain(nxt=nxt):
                    pltpu.make_async_copy(
                        rbs[nxt], acc.at[svs[nxt]], ssem[nxt]).wait()

                @pl.when(cn < NCHK)
                def _pref(nxt=nxt, cn=cn):
                    pltpu.sync_copy(gidx.at[wid].at[pl.ds(cn * R, R)],
                                    gvs[nxt])
                    pltpu.sync_copy(sidx.at[wid].at[pl.ds(cn * R, R)],
                                    svs[nxt])
                    pltpu.async_copy(tab.at[gvs[nxt]], rbs[nxt], gsem[nxt])

                pltpu.make_async_copy(tab.at[gvs[p]], rbs[p], gsem[p]).wait()
                pltpu.async_copy(rbs[p], acc.at[svs[p]], ssem[p], add=True)

            return c2

        lax.fori_loop(0, NCHK // 2, body, 0, unroll=False)
        for p in range(2):
            pltpu.make_async_copy(rbs[p], acc.at[svs[p]], ssem[p]).wait()
        plsc.subcore_barrier()

        @pl.when(sid == 0)
        def _dump():
            pltpu.sync_copy(acc, out.at[cid])

    return apply_k


def _prep(degp, x0p16):
    def body(deg_ref, x0_ref, dinv_ref, yt_ref):
        deg = deg_ref[0, :, 0:1] + deg_ref[1, :, 0:1]
        dinv = jnp.where(deg > 0, lax.rsqrt(deg), 0.0)
        dinv_ref[...] = dinv
        yt_ref[...] = x0_ref[...] * dinv

    return pl.pallas_call(
        body,
        grid=(GR,),
        in_specs=[
            pl.BlockSpec((NC, BR, 16), lambda i: (0, i, 0)),
            pl.BlockSpec((BR, 16), lambda i: (i, 0)),
        ],
        out_specs=[
            pl.BlockSpec((BR, 1), lambda i: (i, 0)),
            pl.BlockSpec((BR, 16), lambda i: (i, 0)),
        ],
        out_shape=[
            jax.ShapeDtypeStruct((NP, 1), jnp.float32),
            jax.ShapeDtypeStruct((NP, 16), jnp.float32),
        ],
    )(degp, x0p16)


def _cheb_step(partials, dinv, prev2):
    nch = len(partials)
    first = prev2 is None
    Cs = [p.shape[-1] for p in partials]

    def body(*refs):
        prefs = refs[:nch]
        dref = refs[nch]
        p2refs = () if first else refs[nch + 1: nch + 1 + nch]
        orefs = refs[nch + 1 + (0 if first else nch):]
        txrefs = orefs[:nch]
        ytrefs = orefs[nch:]
        dv = dref[...]
        for q in range(nch):
            ltx = -(prefs[q][0] + prefs[q][1]) * dv
            tx = ltx if first else 2.0 * ltx - p2refs[q][...]
            txrefs[q][...] = tx
            ytrefs[q][...] = tx * dv

    in_specs = [pl.BlockSpec((NC, BR, C), lambda i: (0, i, 0)) for C in Cs]
    in_specs.append(pl.BlockSpec((BR, 1), lambda i: (i, 0)))
    args = list(partials) + [dinv]
    if not first:
        in_specs += [pl.BlockSpec((BR, C), lambda i: (i, 0)) for C in Cs]
        args += list(prev2)
    out_specs = [pl.BlockSpec((BR, C), lambda i: (i, 0)) for C in Cs] * 2
    out_shape = [jax.ShapeDtypeStruct((NP, C), jnp.float32) for C in Cs] * 2

    outs = pl.pallas_call(
        body, grid=(GR,), in_specs=in_specs, out_specs=out_specs,
        out_shape=out_shape,
    )(*args)
    return list(outs[:nch]), list(outs[nch:])


def _layer_mm(tx_list, W, dinv):
    K = W.shape[0]
    nch = len(tx_list[0])
    Cs = [t.shape[-1] for t in tx_list[0]]
    F = W.shape[1]

    def body(*refs):
        nin = K * nch
        xrefs = refs[:nin]
        wref = refs[nin]
        dref = refs[nin + 1]
        oA, oB, yA, yB = refs[nin + 2: nin + 6]
        acc = jnp.zeros((BR, HID), jnp.float32)
        for k in range(K):
            off = 0
            for q in range(nch):
                xb = xrefs[k * nch + q][...]
                acc = acc + jnp.dot(
                    xb, wref[k, off:off + Cs[q], :],
                    preferred_element_type=jnp.float32)
                off += Cs[q]
        o = jnp.where(acc > 0, acc, 0.5 * acc)
        dv = dref[...]
        oA[...] = o[:, :32]
        oB[...] = o[:, 32:]
        yA[...] = o[:, :32] * dv
        yB[...] = o[:, 32:] * dv

    in_specs = [pl.BlockSpec((BR, C), lambda i: (i, 0))
                for _k in range(K) for C in Cs]
    in_specs.append(pl.BlockSpec((K, F, HID), lambda i: (0, 0, 0)))
    in_specs.append(pl.BlockSpec((BR, 1), lambda i: (i, 0)))
    out_specs = [pl.BlockSpec((BR, 32), lambda i: (i, 0))] * 4
    out_shape = [jax.ShapeDtypeStruct((NP, 32), jnp.float32)] * 4
    args = [t for txs in tx_list for t in txs] + [W, dinv]
    o = pl.pallas_call(
        body, grid=(GR,), in_specs=in_specs, out_specs=out_specs,
        out_shape=out_shape,
    )(*args)
    return [o[0], o[1]], [o[2], o[3]]


def _final(outs, x0p):
    def body(*refs):
        (a1, b1, a2, b2, a3, b3, a4, b4, a5, b5, a6, b6, xref, gref) = refs
        i = pl.program_id(0)

        @pl.when(i == 0)
        def _init():
            gref[...] = jnp.zeros_like(gref)

        xb = xref[...]
        pairs = [(a1, a4), (b1, b4), (a2, a5), (b2, b5), (a3, a6), (b3, b6)]
        accs = []
        for (a, b) in pairs:
            h = a[...] + b[...]
            accs.append(lax.dot_general(
                h, xb, (((0,), (0,)), ((), ())),
                preferred_element_type=jnp.float32))
        g = gref[...] + jnp.concatenate(accs, axis=0)

        @pl.when(i < GR - 1)
        def _store():
            gref[...] = g

        @pl.when(i == GR - 1)
        def _done():
            gref[...] = g * lax.rsqrt(jnp.sum(g * g, axis=1, keepdims=True))

    in_specs = [pl.BlockSpec((BR, 32), lambda i: (i, 0))] * 12
    in_specs.append(pl.BlockSpec((BR, 32), lambda i: (i, 0)))
    return pl.pallas_call(
        body, grid=(GR,), in_specs=in_specs,
        out_specs=pl.BlockSpec((192, 32), lambda i: (0, 0)),
        out_shape=jax.ShapeDtypeStruct((192, 32), jnp.float32),
    )(*outs, x0p)


def _cheb_layer(x_chunks, yt_chunks, W, dinv, gidx, sidx, ztile):
    K = W.shape[0]
    Txs = [list(x_chunks)]
    yt = list(yt_chunks)
    for k in range(1, K):
        partials = [_make_apply(t.shape[-1])(t, gidx, sidx,
                                             ztile[t.shape[-1]])
                    for t in yt]
        prev2 = None if k == 1 else Txs[k - 2]
        tx, yt = _cheb_step(partials, dinv, prev2)
        Txs.append(tx)
    return _layer_mm(Txs, W, dinv)


def kernel(pos, x, batch, edge_index, W1, W2, W3, W4, W5, W6):
    src = edge_index[0].astype(jnp.int32)
    dst = edge_index[1].astype(jnp.int32)

    nzr = NP - N_NODES
    srcg = jnp.where(src == dst, N_NODES + src % nzr, src)
    npad = EPAD - N_EDGES
    padg = N_NODES + jnp.arange(npad, dtype=jnp.int32) % nzr
    pads = jnp.arange(npad, dtype=jnp.int32) % N_NODES
    gidx = jnp.concatenate([srcg, padg]).reshape(NW, EPT)
    sidx_dst = jnp.concatenate([dst, pads]).reshape(NW, EPT)
    sidx_src = jnp.concatenate([src, pads]).reshape(NW, EPT)

    x0 = jnp.concatenate([pos, x], axis=1).astype(jnp.float32)
    x0p = jnp.pad(x0, ((0, NP - N_NODES), (0, 32 - 9)))
    x0p16 = x0p[:, :16]
    e0 = jnp.pad(jnp.ones((N_NODES, 1), jnp.float32),
                 ((0, NP - N_NODES), (0, 15)))
    zt = {16: jnp.zeros((CH, 16), jnp.float32),
          32: jnp.zeros((CH, 32), jnp.float32)}
    W1p = jnp.pad(W1, ((0, 0), (0, 7), (0, 0)))
    W4p = jnp.pad(W4, ((0, 0), (0, 7), (0, 0)))

    degp = _make_apply(16)(e0, gidx, sidx_src, zt[16])
    dinv, ytab0 = _prep(degp, x0p16)

    run = functools.partial(_cheb_layer, dinv=dinv, gidx=gidx,
                            sidx=sidx_dst, ztile=zt)
    out1, yt1 = run([x0p16], [ytab0], W1p)
    out2, yt2 = run(out1, yt1, W2)
    out3, _ = run(out2, yt2, W3)
    out4, yt4 = run([x0p16], [ytab0], W4p)
    out5, yt5 = run(out4, yt4, W5)
    out6, _ = run(out5, yt5, W6)

    G = _final(out1 + out2 + out3 + out4 + out5 + out6, x0p)
    return G[:, :9].reshape(1, 192, 9)


# speedup vs baseline: 2.0759x; 1.1615x over previous
import functools

import jax
import jax.numpy as jnp
from jax import lax
from jax.experimental import pallas as pl
from jax.experimental.pallas import tpu as pltpu
from jax.experimental.pallas import tpu_sc as plsc

N_NODES = 50000
N_EDGES = 800000
HID = 64
NP = 50048
NC, NS = 2, 16
CH = 128
EPT = 25600
EPT2 = 51200
R = 400
EPAD = 819200
BR = 2176
GR = NP // BR


def _stream_loop(tab, gidx, sidx, wid, ept, gvs, svs, rbs, acc, gsem, ssem):
    nchk = ept // R
    pltpu.sync_copy(gidx.at[wid].at[pl.ds(0, R)], gvs[0])
    pltpu.sync_copy(sidx.at[wid].at[pl.ds(0, R)], svs[0])
    pltpu.async_copy(tab.at[gvs[0]], rbs[0], gsem[0])

    def body(i, c2):
        for p in range(2):
            c = i * 2 + p
            cn = c + 1
            nxt = 1 - p

            @pl.when((cn < nchk) & (c >= 1))
            def _drain(nxt=nxt):
                pltpu.make_async_copy(
                    rbs[nxt], acc.at[svs[nxt]], ssem[nxt]).wait()

            @pl.when(cn < nchk)
            def _pref(nxt=nxt, cn=cn):
                pltpu.sync_copy(gidx.at[wid].at[pl.ds(cn * R, R)], gvs[nxt])
                pltpu.sync_copy(sidx.at[wid].at[pl.ds(cn * R, R)], svs[nxt])
                pltpu.async_copy(tab.at[gvs[nxt]], rbs[nxt], gsem[nxt])

            pltpu.make_async_copy(tab.at[gvs[p]], rbs[p], gsem[p]).wait()
            pltpu.async_copy(rbs[p], acc.at[svs[p]], ssem[p], add=True)

        return c2

    lax.fori_loop(0, nchk // 2, body, 0, unroll=False)
    for p in range(2):
        pltpu.make_async_copy(rbs[p], acc.at[svs[p]], ssem[p]).wait()


def _zero_acc(ztile, acc, sid, C):
    nzt = NP // CH

    def zero_body(i, carry):
        t = i * NS + sid

        @pl.when(t < nzt)
        def _():
            pltpu.sync_copy(ztile, acc.at[pl.ds(t * CH, CH)])

        return carry

    lax.fori_loop(0, (nzt + NS - 1) // NS, zero_body, 0, unroll=False)


def _sc_scratch(C):
    return [
        pltpu.VMEM((R,), jnp.int32),
        pltpu.VMEM((R,), jnp.int32),
        pltpu.VMEM((R,), jnp.int32),
        pltpu.VMEM((R,), jnp.int32),
        pltpu.VMEM((R, C), jnp.float32),
        pltpu.VMEM((R, C), jnp.float32),
        pltpu.VMEM_SHARED((NP, C), jnp.float32),
    ] + [pltpu.SemaphoreType.DMA] * 4


@functools.lru_cache(None)
def _make_apply16():
    mesh = plsc.VectorSubcoreMesh(core_axis_name="c", subcore_axis_name="s")

    @functools.partial(
        pl.kernel,
        out_type=jax.ShapeDtypeStruct((NC, NP, 16), jnp.float32),
        mesh=mesh,
        scratch_types=_sc_scratch(16),
        compiler_params=pltpu.CompilerParams(use_tc_tiling_on_sc=False),
    )
    def apply_k(tab, gidx, sidx, ztile, out, gv0, gv1, sv0, sv1, rb0, rb1,
                acc, g0, g1, t0, t1):
        cid = lax.axis_index("c")
        sid = lax.axis_index("s")
        wid = sid * NC + cid
        _zero_acc(ztile, acc, sid, 16)
        plsc.subcore_barrier()
        _stream_loop(tab, gidx, sidx, wid, EPT, (gv0, gv1), (sv0, sv1),
                     (rb0, rb1), acc, (g0, g1), (t0, t1))
        plsc.subcore_barrier()

        @pl.when(sid == 0)
        def _dump():
            pltpu.sync_copy(acc, out.at[cid])

    return apply_k


@functools.lru_cache(None)
def _make_apply32():
    mesh = plsc.VectorSubcoreMesh(core_axis_name="c", subcore_axis_name="s")

    @functools.partial(
        pl.kernel,
        out_type=jax.ShapeDtypeStruct((NC, NP, 32), jnp.float32),
        mesh=mesh,
        scratch_types=_sc_scratch(32),
        compiler_params=pltpu.CompilerParams(use_tc_tiling_on_sc=False),
    )
    def apply_k(tab, gidx, sidx, ztile, out, gv0, gv1, sv0, sv1, rb0, rb1,
                acc, g0, g1, t0, t1):
        cid = lax.axis_index("c")
        sid = lax.axis_index("s")
        _zero_acc(ztile, acc, sid, 32)
        plsc.subcore_barrier()
        _stream_loop(tab, gidx.at[cid], sidx, sid, EPT2, (gv0, gv1),
                     (sv0, sv1), (rb0, rb1), acc, (g0, g1), (t0, t1))
        plsc.subcore_barrier()

        @pl.when(sid == 0)
        def _dump():
            pltpu.sync_copy(acc, out.at[cid])

    return apply_k


def _prep(degp, x0p16):
    def body(deg_ref, x0_ref, dinv_ref, yt_ref):
        deg = deg_ref[0, :, 0:1] + deg_ref[1, :, 0:1]
        dinv = jnp.where(deg > 0, lax.rsqrt(deg), 0.0)
        dinv_ref[...] = dinv
        yt_ref[...] = x0_ref[...] * dinv

    return pl.pallas_call(
        body,
        grid=(GR,),
        in_specs=[
            pl.BlockSpec((NC, BR, 16), lambda i: (0, i, 0)),
            pl.BlockSpec((BR, 16), lambda i: (i, 0)),
        ],
        out_specs=[
            pl.BlockSpec((BR, 1), lambda i: (i, 0)),
            pl.BlockSpec((BR, 16), lambda i: (i, 0)),
        ],
        out_shape=[
            jax.ShapeDtypeStruct((NP, 1), jnp.float32),
            jax.ShapeDtypeStruct((NP, 16), jnp.float32),
        ],
    )(degp, x0p16)


def _cheb_step16(partials, dinv, prev2):
    first = prev2 is None

    def body(*refs):
        pref, dref = refs[0], refs[1]
        p2 = None if first else refs[2]
        txr, ytr = refs[-2], refs[-1]
        dv = dref[...]
        ltx = -(pref[0] + pref[1]) * dv
        tx = ltx if first else 2.0 * ltx - p2[...]
        txr[...] = tx
        ytr[...] = tx * dv

    in_specs = [
        pl.BlockSpec((NC, BR, 16), lambda i: (0, i, 0)),
        pl.BlockSpec((BR, 1), lambda i: (i, 0)),
    ]
    args = [partials, dinv]
    if not first:
        in_specs.append(pl.BlockSpec((BR, 16), lambda i: (i, 0)))
        args.append(prev2)
    return pl.pallas_call(
        body, grid=(GR,), in_specs=in_specs,
        out_specs=[pl.BlockSpec((BR, 16), lambda i: (i, 0))] * 2,
        out_shape=[jax.ShapeDtypeStruct((NP, 16), jnp.float32)] * 2,
    )(*args)


def _cheb_step32(sums, dinv, prev2):
    first = prev2 is None

    def body(*refs):
        sref, dref = refs[0], refs[1]
        p2 = None if first else refs[2]
        txr, ytr = refs[-2], refs[-1]
        dv = dref[...]
        for q in range(NC):
            ltx = -sref[q] * dv
            tx = ltx if first else 2.0 * ltx - p2[q]
            txr[q] = tx
            ytr[q] = tx * dv

    spec2 = pl.BlockSpec((NC, BR, 32), lambda i: (0, i, 0))
    in_specs = [spec2, pl.BlockSpec((BR, 1), lambda i: (i, 0))]
    args = [sums, dinv]
    if not first:
        in_specs.append(spec2)
        args.append(prev2)
    return pl.pallas_call(
        body, grid=(GR,), in_specs=in_specs,
        out_specs=[spec2] * 2,
        out_shape=[jax.ShapeDtypeStruct((NC, NP, 32), jnp.float32)] * 2,
    )(*args)


def _layer_mm16(tx_list, W, dinv):
    K = len(tx_list)

    def body(*refs):
        xrefs = refs[:K]
        wref, dref, oref, yref = refs[K:]
        acc = jnp.zeros((BR, HID), jnp.float32)
        for k in range(K):
            acc = acc + jnp.dot(xrefs[k][...], wref[k],
                                preferred_element_type=jnp.float32)
        o = jnp.where(acc > 0, acc, 0.5 * acc)
        dv = dref[...]
        for q in range(NC):
            ob = o[:, 32 * q:32 * q + 32]
            oref[q] = ob
            yref[q] = ob * dv

    in_specs = [pl.BlockSpec((BR, 16), lambda i: (i, 0))] * K
    in_specs.append(pl.BlockSpec((K, 16, HID), lambda i: (0, 0, 0)))
    in_specs.append(pl.BlockSpec((BR, 1), lambda i: (i, 0)))
    spec2 = pl.BlockSpec((NC, BR, 32), lambda i: (0, i, 0))
    return pl.pallas_call(
        body, grid=(GR,), in_specs=in_specs, out_specs=[spec2] * 2,
        out_shape=[jax.ShapeDtypeStruct((NC, NP, 32), jnp.float32)] * 2,
    )(*tx_list, W, dinv)


def _layer_mm32(tx_list, W, dinv):
    K = len(tx_list)

    def body(*refs):
        xrefs = refs[:K]
        wref, dref, oref, yref = refs[K:]
        acc = jnp.zeros((BR, HID), jnp.float32)
        for k in range(K):
            for q in range(NC):
                acc = acc + jnp.dot(
                    xrefs[k][q], wref[k, 32 * q:32 * q + 32, :],
                    preferred_element_type=jnp.float32)
        o = jnp.where(acc > 0, acc, 0.5 * acc)
        dv = dref[...]
        for q in range(NC):
            ob = o[:, 32 * q:32 * q + 32]
            oref[q] = ob
            yref[q] = ob * dv

    spec2 = pl.BlockSpec((NC, BR, 32), lambda i: (0, i, 0))
    in_specs = [spec2] * K
    in_specs.append(pl.BlockSpec((K, HID, HID), lambda i: (0, 0, 0)))
    in_specs.append(pl.BlockSpec((BR, 1), lambda i: (i, 0)))
    return pl.pallas_call(
        body, grid=(GR,), in_specs=in_specs, out_specs=[spec2] * 2,
        out_shape=[jax.ShapeDtypeStruct((NC, NP, 32), jnp.float32)] * 2,
    )(*tx_list, W, dinv)


def _final(outs, x0p):
    def body(*refs):
        (o1, o2, o3, o4, o5, o6, xref, gref) = refs
        i = pl.program_id(0)

        @pl.when(i == 0)
        def _init():
            gref[...] = jnp.zeros_like(gref)

        xb = xref[...]
        accs = []
        for (a, b) in ((o1, o4), (o2, o5), (o3, o6)):
            for q in range(NC):
                h = a[q] + b[q]
                accs.append(lax.dot_general(
                    h, xb, (((0,), (0,)), ((), ())),
                    preferred_element_type=jnp.float32))
        g = gref[...] + jnp.concatenate(accs, axis=0)

        @pl.when(i < GR - 1)
        def _store():
            gref[...] = g

        @pl.when(i == GR - 1)
        def _done():
            gref[...] = g * lax.rsqrt(jnp.sum(g * g, axis=1, keepdims=True))

    spec2 = pl.BlockSpec((NC, BR, 32), lambda i: (0, i, 0))
    in_specs = [spec2] * 6 + [pl.BlockSpec((BR, 32), lambda i: (i, 0))]
    return pl.pallas_call(
        body, grid=(GR,), in_specs=in_specs,
        out_specs=pl.BlockSpec((192, 32), lambda i: (0, 0)),
        out_shape=jax.ShapeDtypeStruct((192, 32), jnp.float32),
    )(*outs, x0p)


def _run_pair(chain_a, chain_b, dinv, gidx2o, sidx2, zt32):
    apply32 = _make_apply32()
    chains = [dict(Txs=[c[0]], yt=c[1], W=c[2]) for c in (chain_a, chain_b)]
    kmax = max(c["W"].shape[0] for c in chains)
    for k in range(1, kmax):
        for ch in chains:
            if k < ch["W"].shape[0]:
                tab = ch["yt"].reshape(NC * NP, 32)
                sums = apply32(tab, gidx2o, sidx2, zt32)
                prev2 = None if k == 1 else ch["Txs"][k - 2]
                tx, yt = _cheb_step32(sums, dinv, prev2)
                ch["Txs"].append(tx)
                ch["yt"] = yt
    return [_layer_mm32(ch["Txs"], ch["W"], dinv) for ch in chains]


def kernel(pos, x, batch, edge_index, W1, W2, W3, W4, W5, W6):
    src = edge_index[0].astype(jnp.int32)
    dst = edge_index[1].astype(jnp.int32)

    nzr = NP - N_NODES
    srcg = jnp.where(src == dst, N_NODES + src % nzr, src)
    npad = EPAD - N_EDGES
    padg = N_NODES + jnp.arange(npad, dtype=jnp.int32) % nzr
    pads = jnp.arange(npad, dtype=jnp.int32) % N_NODES
    gflat = jnp.concatenate([srcg, padg])
    sflat_dst = jnp.concatenate([dst, pads])
    gidx = gflat.reshape(NC * NS, EPT)
    sidx_dst = sflat_dst.reshape(NC * NS, EPT)
    sidx_src = jnp.concatenate([src, pads]).reshape(NC * NS, EPT)
    g2 = gflat.reshape(NS, EPT2)
    gidx2o = jnp.stack([g2, g2 + NP])
    sidx2 = sflat_dst.reshape(NS, EPT2)

    x0 = jnp.concatenate([pos, x], axis=1).astype(jnp.float32)
    x0p = jnp.pad(x0, ((0, NP - N_NODES), (0, 32 - 9)))
    x0p16 = x0p[:, :16]
    e0 = jnp.pad(jnp.ones((N_NODES, 1), jnp.float32),
                 ((0, NP - N_NODES), (0, 15)))
    zt16 = jnp.zeros((CH, 16), jnp.float32)
    zt32 = jnp.zeros((CH, 32), jnp.float32)
    W1p = jnp.pad(W1, ((0, 0), (0, 7), (0, 0)))
    W4p = jnp.pad(W4, ((0, 0), (0, 7), (0, 0)))

    apply16 = _make_apply16()
    degp = apply16(e0, gidx, sidx_src, zt16)
    dinv, ytab0 = _prep(degp, x0p16)

    Txs = [x0p16]
    yt = ytab0
    for k in range(1, 8):
        part = apply16(yt, gidx, sidx_dst, zt16)
        prev2 = None if k == 1 else Txs[k - 2]
        tx, yt = _cheb_step16(part, dinv, prev2)
        Txs.append(tx)
    out1, y1 = _layer_mm16(Txs[:4], W1p, dinv)
    out4, y4 = _layer_mm16(Txs, W4p, dinv)

    (out2, y2), (out5, y5) = _run_pair((out1, y1, W2), (out4, y4, W5),
                                       dinv, gidx2o, sidx2, zt32)
    (out3, _), (out6, _) = _run_pair((out2, y2, W3), (out5, y5, W6),
                                     dinv, gidx2o, sidx2, zt32)

    G = _final([out1, out2, out3, out4, out5, out6], x0p)
    return G[:, :9].reshape(1, 192, 9)
